# baseline (speedup 1.0000x reference)
import functools

import jax
import jax.numpy as jnp
from jax import lax
from jax.experimental import pallas as pl
from jax.experimental.pallas import tpu as pltpu
from jax.experimental.pallas import tpu_sc as plsc

N_NODES = 10000
N_EDGES = 320000
D = 128

NC = 2
NS = 16
HALF = 5120
TRASH = HALF
AROWS = HALF + 8
CH = 128
HCH = 80
NCH = 2 * HCH
EPS = NCH * CH
EPAD = NS * EPS
RPS = HALF // NS


def _sc_agg_body(src_hbm, dst_hbm, x_hbm, out_hbm,
                 srcv, dstv, r0, r1, r2, r3, zbuf, aggsh,
                 gs0, gs1, gs2, gs3, ss0, ss1, ss2, ss3):
    rows = (r0, r1, r2, r3)
    gsem = (gs0, gs1, gs2, gs3)
    ssem = (ss0, ss1, ss2, ss3)
    cid = lax.axis_index("c")
    sid = lax.axis_index("s")
    lo = cid * HALF

    for r in range(8):
        for c in range(D // 16):
            zbuf[r, pl.ds(c * 16, 16)] = jnp.zeros((16,), jnp.float32)

    def _zfill(z, carry):
        pltpu.sync_copy(zbuf, aggsh.at[pl.ds(sid * RPS + z * 8, 8)])
        return carry
    lax.fori_loop(0, RPS // 8, _zfill, 0)

    @pl.when(sid == NS - 1)
    def _zero_trash():
        pltpu.sync_copy(zbuf, aggsh.at[pl.ds(HALF, 8)])

    plsc.subcore_barrier()

    def _remap(i):
        for j in range(CH // 16):
            t = dstv[i, pl.ds(j * 16, 16)] - lo
            oob = (t < 0) | (t >= HALF)
            dstv[i, pl.ds(j * 16, 16)] = jnp.where(oob, TRASH, t)

    for h in range(2):
        pltpu.sync_copy(src_hbm.at[sid, h], srcv)
        pltpu.sync_copy(dst_hbm.at[sid, h], dstv)

        pltpu.async_copy(x_hbm.at[srcv.at[0]], r0, gs0)
        pltpu.async_copy(x_hbm.at[srcv.at[1]], r1, gs1)

        def _quad(t, carry):
            for k in range(4):
                j = 4 * t + k
                pltpu.make_async_copy(
                    x_hbm.at[srcv.at[j]], rows[k], gsem[k]).wait()
                _remap(j)
                pltpu.async_copy(
                    rows[k], aggsh.at[dstv.at[j]], ssem[k], add=True)
                nk = (k + 2) % 4

                @pl.when(j >= 2)
                def _wait_scatter():
                    pltpu.make_async_copy(
                        rows[nk], aggsh.at[dstv.at[j - 2]], ssem[nk]).wait()

                @pl.when(j < HCH - 2)
                def _prefetch():
                    pltpu.async_copy(
                        x_hbm.at[srcv.at[j + 2]], rows[nk], gsem[nk])
            return carry

        lax.fori_loop(0, HCH // 4, _quad, 0)
        pltpu.make_async_copy(r2, aggsh.at[dstv.at[HCH - 2]], ss2).wait()
        pltpu.make_async_copy(r3, aggsh.at[dstv.at[HCH - 1]], ss3).wait()

    plsc.subcore_barrier()
    pltpu.sync_copy(aggsh.at[pl.ds(sid * RPS, RPS)],
                    out_hbm.at[pl.ds(cid * HALF + sid * RPS, RPS)])


_sc_agg = functools.partial(
    pl.kernel,
    out_type=jax.ShapeDtypeStruct((NC * HALF, D), jnp.float32),
    mesh=plsc.VectorSubcoreMesh(core_axis_name="c", subcore_axis_name="s"),
    scratch_types=[
        pltpu.VMEM((HCH, CH), jnp.int32),
        pltpu.VMEM((HCH, CH), jnp.int32),
        pltpu.VMEM((CH, D), jnp.float32),
        pltpu.VMEM((CH, D), jnp.float32),
        pltpu.VMEM((CH, D), jnp.float32),
        pltpu.VMEM((CH, D), jnp.float32),
        pltpu.VMEM((8, D), jnp.float32),
        pltpu.VMEM_SHARED((AROWS, D), jnp.float32),
        pltpu.SemaphoreType.DMA,
        pltpu.SemaphoreType.DMA,
        pltpu.SemaphoreType.DMA,
        pltpu.SemaphoreType.DMA,
        pltpu.SemaphoreType.DMA,
        pltpu.SemaphoreType.DMA,
        pltpu.SemaphoreType.DMA,
        pltpu.SemaphoreType.DMA,
    ],
    name="sc_edge_segment_sum",
)(_sc_agg_body)


def _mlp_body(x_ref, agg_ref, w1a_ref, w1b_ref, b1_ref,
              gamma_ref, beta_ref, w2_ref, b2_ref, out_ref):
    h = jnp.dot(x_ref[...], w1a_ref[...], preferred_element_type=jnp.float32)
    h = h + jnp.dot(agg_ref[:N_NODES], w1b_ref[...],
                    preferred_element_type=jnp.float32)
    h = jnp.maximum(h + b1_ref[...], 0.0)
    mean = jnp.mean(h, axis=0, keepdims=True)
    cen = h - mean
    var = jnp.mean(cen * cen, axis=0, keepdims=True)
    hn = cen * (lax.rsqrt(var + 1e-5) * gamma_ref[...]) + beta_ref[...]
    out_ref[...] = (
        jnp.dot(hn, w2_ref[...], preferred_element_type=jnp.float32)
        + b2_ref[...])


def kernel(x, edge_index, W1, b1, gamma, beta, W2, b2):
    npad = EPAD - N_EDGES
    src = jnp.concatenate(
        [edge_index[0], jnp.zeros((npad,), jnp.int32)]
    ).reshape(NS, 2, HCH, CH)
    dst = jnp.concatenate(
        [edge_index[1], jnp.full((npad,), N_NODES, jnp.int32)]
    ).reshape(NS, 2, HCH, CH)
    agg = _sc_agg(src, dst, x)
    return pl.pallas_call(
        _mlp_body,
        out_shape=jax.ShapeDtypeStruct((N_NODES, D), jnp.float32),
    )(x, agg, W1[:D], W1[D:], b1.reshape(1, D),
      gamma.reshape(1, D), beta.reshape(1, D), W2, b2.reshape(1, D))


# speedup vs baseline: 1.0056x; 1.0056x over previous
import functools

import jax
import jax.numpy as jnp
from jax import lax
from jax.experimental import pallas as pl
from jax.experimental.pallas import tpu as pltpu
from jax.experimental.pallas import tpu_sc as plsc

N_NODES = 10000
N_EDGES = 320000
D = 128

NC = 2
NS = 16
HALF = 5120
TRASH = HALF
AROWS = HALF + 8
CH = 128
HCH = 80
NCH = 2 * HCH
EPS = NCH * CH
EPAD = NS * EPS
RPS = HALF // NS


def _sc_agg_body(src_hbm, dst_hbm, x_hbm, out_hbm,
                 srcv, dstv, r0, r1, r2, r3, zbuf, aggsh,
                 gs0, gs1, gs2, gs3, ss0, ss1, ss2, ss3):
    rows = (r0, r1, r2, r3)
    gsem = (gs0, gs1, gs2, gs3)
    ssem = (ss0, ss1, ss2, ss3)
    cid = lax.axis_index("c")
    sid = lax.axis_index("s")
    lo = cid * HALF

    for r in range(8):
        for c in range(D // 16):
            zbuf[r, pl.ds(c * 16, 16)] = jnp.zeros((16,), jnp.float32)

    def _zfill(z, carry):
        pltpu.sync_copy(zbuf, aggsh.at[pl.ds(sid * RPS + z * 8, 8)])
        return carry
    lax.fori_loop(0, RPS // 8, _zfill, 0)

    @pl.when(sid == NS - 1)
    def _zero_trash():
        pltpu.sync_copy(zbuf, aggsh.at[pl.ds(HALF, 8)])

    plsc.subcore_barrier()

    def _remap(i):
        for j in range(CH // 16):
            t = dstv[i, pl.ds(j * 16, 16)] - lo
            oob = (t < 0) | (t >= HALF)
            dstv[i, pl.ds(j * 16, 16)] = jnp.where(oob, TRASH, t)

    for h in range(2):
        pltpu.sync_copy(src_hbm.at[sid, h], srcv)
        pltpu.sync_copy(dst_hbm.at[sid, h], dstv)

        pltpu.async_copy(x_hbm.at[srcv.at[0]], r0, gs0)
        pltpu.async_copy(x_hbm.at[srcv.at[1]], r1, gs1)

        def _quad(t, carry):
            for k in range(4):
                j = 4 * t + k
                pltpu.make_async_copy(
                    x_hbm.at[srcv.at[j]], rows[k], gsem[k]).wait()
                _remap(j)
                pk = (k + 3) % 4
                nk = (k + 2) % 4

                @pl.when(j >= 1)
                def _wait_scatter():
                    pltpu.make_async_copy(
                        rows[pk], aggsh.at[dstv.at[j - 1]], ssem[pk]).wait()

                pltpu.async_copy(
                    rows[k], aggsh.at[dstv.at[j]], ssem[k], add=True)

                @pl.when(j < HCH - 2)
                def _prefetch():
                    pltpu.async_copy(
                        x_hbm.at[srcv.at[j + 2]], rows[nk], gsem[nk])
            return carry

        lax.fori_loop(0, HCH // 4, _quad, 0)
        pltpu.make_async_copy(r3, aggsh.at[dstv.at[HCH - 1]], ss3).wait()

    plsc.subcore_barrier()
    pltpu.sync_copy(aggsh.at[pl.ds(sid * RPS, RPS)],
                    out_hbm.at[pl.ds(cid * HALF + sid * RPS, RPS)])


_sc_agg = functools.partial(
    pl.kernel,
    out_type=jax.ShapeDtypeStruct((NC * HALF, D), jnp.float32),
    mesh=plsc.VectorSubcoreMesh(core_axis_name="c", subcore_axis_name="s"),
    scratch_types=[
        pltpu.VMEM((HCH, CH), jnp.int32),
        pltpu.VMEM((HCH, CH), jnp.int32),
        pltpu.VMEM((CH, D), jnp.float32),
        pltpu.VMEM((CH, D), jnp.float32),
        pltpu.VMEM((CH, D), jnp.float32),
        pltpu.VMEM((CH, D), jnp.float32),
        pltpu.VMEM((8, D), jnp.float32),
        pltpu.VMEM_SHARED((AROWS, D), jnp.float32),
        pltpu.SemaphoreType.DMA,
        pltpu.SemaphoreType.DMA,
        pltpu.SemaphoreType.DMA,
        pltpu.SemaphoreType.DMA,
        pltpu.SemaphoreType.DMA,
        pltpu.SemaphoreType.DMA,
        pltpu.SemaphoreType.DMA,
        pltpu.SemaphoreType.DMA,
    ],
    name="sc_edge_segment_sum",
)(_sc_agg_body)


def _mlp_body(x_ref, agg_ref, w1a_ref, w1b_ref, b1_ref,
              gamma_ref, beta_ref, w2_ref, b2_ref, out_ref):
    h = jnp.dot(x_ref[...], w1a_ref[...], preferred_element_type=jnp.float32)
    h = h + jnp.dot(agg_ref[:N_NODES], w1b_ref[...],
                    preferred_element_type=jnp.float32)
    h = jnp.maximum(h + b1_ref[...], 0.0)
    mean = jnp.mean(h, axis=0, keepdims=True)
    cen = h - mean
    var = jnp.mean(cen * cen, axis=0, keepdims=True)
    hn = cen * (lax.rsqrt(var + 1e-5) * gamma_ref[...]) + beta_ref[...]
    out_ref[...] = (
        jnp.dot(hn, w2_ref[...], preferred_element_type=jnp.float32)
        + b2_ref[...])


def kernel(x, edge_index, W1, b1, gamma, beta, W2, b2):
    npad = EPAD - N_EDGES
    src = jnp.concatenate(
        [edge_index[0], jnp.zeros((npad,), jnp.int32)]
    ).reshape(NS, 2, HCH, CH)
    dst = jnp.concatenate(
        [edge_index[1], jnp.full((npad,), N_NODES, jnp.int32)]
    ).reshape(NS, 2, HCH, CH)
    agg = _sc_agg(src, dst, x)
    return pl.pallas_call(
        _mlp_body,
        out_shape=jax.ShapeDtypeStruct((N_NODES, D), jnp.float32),
    )(x, agg, W1[:D], W1[D:], b1.reshape(1, D),
      gamma.reshape(1, D), beta.reshape(1, D), W2, b2.reshape(1, D))
